# baseline (device time: 17141 ns/iter reference)
import jax
import jax.numpy as jnp
from jax import lax
from jax.experimental import pallas as pl
from jax.experimental.pallas import tpu as pltpu

N_DEV = 4
TAPS = 4
HALO = TAPS - 1


def kernel(x, k):
    b, s, c = x.shape

    def body(x_hbm, x_ref, k_ref, out_ref, send_buf, halo_buf,
             send_sem, recv_sem, tail_sem):
        i = pl.program_id(0)
        my_pos = lax.axis_index("i")
        left = lax.rem(my_pos + N_DEV - 1, N_DEV)
        right = lax.rem(my_pos + 1, N_DEV)
        k_val = k_ref[...]

        rdma = pltpu.make_async_remote_copy(
            src_ref=send_buf,
            dst_ref=halo_buf,
            send_sem=send_sem,
            recv_sem=recv_sem,
            device_id=(right,),
            device_id_type=pl.DeviceIdType.MESH,
        )

        @pl.when(i == 0)
        def _start():
            barrier_sem = pltpu.get_barrier_semaphore()
            for nbr in (left, right):
                pl.semaphore_signal(
                    barrier_sem, inc=1,
                    device_id=(nbr,), device_id_type=pl.DeviceIdType.MESH,
                )
            pl.semaphore_wait(barrier_sem, 2)
            tail = pltpu.make_async_copy(
                x_hbm.at[:, pl.ds(s - HALO, HALO), :], send_buf, tail_sem
            )
            tail.start()
            tail.wait()
            rdma.start()

        xb = x_ref[...]
        xp = jnp.concatenate(
            [jnp.zeros((1, HALO, c), xb.dtype), xb], axis=1
        )
        acc = xb * k_val[TAPS - 1, :][None, None, :]
        for t in range(TAPS - 1):
            acc += xp[:, t:t + s, :] * k_val[t, :][None, None, :]
        out_ref[...] = acc * (1.0 / (1.0 + jnp.exp(-acc)))

        @pl.when(i == 0)
        def _wait():
            rdma.wait()

        halo = halo_buf[pl.ds(i, 1), :, :]
        halo = jnp.where(my_pos == 0, jnp.zeros_like(halo), halo)
        head = jnp.concatenate([halo, xb[:, :HALO, :]], axis=1)
        acc0 = head[:, 0:HALO, :] * k_val[0, :][None, None, :]
        for t in range(1, TAPS):
            acc0 += head[:, t:t + HALO, :] * k_val[t, :][None, None, :]
        out_ref[:, :HALO, :] = acc0 * (1.0 / (1.0 + jnp.exp(-acc0)))

    return pl.pallas_call(
        body,
        grid=(b,),
        out_shape=jax.ShapeDtypeStruct((b, s, c), x.dtype),
        in_specs=[
            pl.BlockSpec(memory_space=pl.ANY),
            pl.BlockSpec((1, s, c), lambda i: (i, 0, 0)),
            pl.BlockSpec((TAPS, c), lambda i: (0, 0)),
        ],
        out_specs=pl.BlockSpec((1, s, c), lambda i: (i, 0, 0)),
        scratch_shapes=[
            pltpu.VMEM((b, HALO, c), x.dtype),
            pltpu.VMEM((b, HALO, c), x.dtype),
            pltpu.SemaphoreType.DMA,
            pltpu.SemaphoreType.DMA,
            pltpu.SemaphoreType.DMA,
        ],
        compiler_params=pltpu.CompilerParams(collective_id=0),
    )(x, x, k)
